# Initial kernel scaffold; baseline (speedup 1.0000x reference)
import jax
import jax.numpy as jnp
from jax.experimental import pallas as pl


def kernel(product_table, qtype_table, ctype_table, W1, b1, W2, b2, Wp, bp, query_ids, query_types):
    raise NotImplementedError("write your pallas kernel here")


# speedup vs baseline: 1.9029x; 1.9029x over previous
import functools

import jax
import jax.numpy as jnp
from jax import lax
from jax.experimental import pallas as pl
from jax.experimental.pallas import tpu as pltpu
from jax.experimental.pallas import tpu_sc as plsc

_B = 16384
_D = 128
_T = 64
_NT = 1000
_K = 8
_BM = 256


def _sc_gather_embeddings(ptable, ttable, qids2d, qtys2d):
    mesh = plsc.VectorSubcoreMesh(core_axis_name="c", subcore_axis_name="s")
    nw = mesh.num_cores * mesh.num_subcores
    rows = qids2d.shape[0]
    rpw = rows // nw

    @functools.partial(
        pl.kernel,
        out_type=(
            jax.ShapeDtypeStruct((_B, _D), jnp.float32),
            jax.ShapeDtypeStruct((_B, _D), jnp.float32),
        ),
        mesh=mesh,
        scratch_types=[
            pltpu.VMEM((rpw, 128), jnp.int32),
            pltpu.VMEM((rpw, 128), jnp.int32),
            pltpu.VMEM((2, 128, _D), jnp.float32),
            pltpu.VMEM((2, 128, _D), jnp.float32),
            pltpu.SemaphoreType.DMA,
        ],
    )
    def k(pt, tt, qi, qy, oq, ot, qi_v, qy_v, qbuf, tbuf, sem):
        wid = lax.axis_index("s") * mesh.num_cores + lax.axis_index("c")
        rbase = wid * rpw
        pltpu.sync_copy(qi.at[pl.ds(rbase, rpw)], qi_v)
        pltpu.sync_copy(qy.at[pl.ds(rbase, rpw)], qy_v)
        for j in range(rpw):
            s = j % 2
            cq = pltpu.async_copy(pt.at[qi_v.at[j]], qbuf.at[s], sem)
            ct = pltpu.async_copy(tt.at[qy_v.at[j]], tbuf.at[s], sem)
            cq.wait()
            ct.wait()
            obase = (rbase + j) * 128
            pltpu.sync_copy(qbuf.at[s], oq.at[pl.ds(obase, 128)])
            pltpu.sync_copy(tbuf.at[s], ot.at[pl.ds(obase, 128)])

    return k(ptable, ttable, qids2d, qtys2d)


def _sc_gather_ctypes(ttable, idx2d):
    mesh = plsc.VectorSubcoreMesh(core_axis_name="c", subcore_axis_name="s")
    nw = mesh.num_cores * mesh.num_subcores
    rows = idx2d.shape[0]
    rpw = rows // nw
    nbk = rows * 128

    @functools.partial(
        pl.kernel,
        out_type=jax.ShapeDtypeStruct((nbk, _D), jnp.float32),
        mesh=mesh,
        scratch_types=[
            pltpu.VMEM((rpw, 128), jnp.int32),
            pltpu.VMEM((2, 128, _D), jnp.float32),
            pltpu.SemaphoreType.DMA,
        ],
    )
    def k(tt, idx, out, idx_v, buf, sem):
        wid = lax.axis_index("s") * mesh.num_cores + lax.axis_index("c")
        rbase = wid * rpw
        pltpu.sync_copy(idx.at[pl.ds(rbase, rpw)], idx_v)

        def body(c, carry):
            cp = pltpu.async_copy(tt.at[idx_v.at[c]], buf.at[0], sem)
            cp.wait()
            pltpu.sync_copy(buf.at[0], out.at[pl.ds((rbase + c) * 128, 128)])
            return carry

        lax.fori_loop(0, rpw, body, 0)

    return k(ttable, idx2d)


def _tc_types_body(qt_ref, qe_ref, w1_ref, b1_ref, w2_ref, b2_ref, ctt_ref,
                   wpa_ref, bp_ref, sims_ref, topk_ref, qp_ref):
    qt = qt_ref[...]
    h = jnp.maximum(qt @ w1_ref[...] + b1_ref[...], 0.0)
    cb = h @ w2_ref[...] + b2_ref[...]
    s = jnp.dot(cb, ctt_ref[...], preferred_element_type=jnp.float32)
    sims_ref[...] = s
    iota = lax.broadcasted_iota(jnp.int32, s.shape, 1)
    work = s
    cols = []
    for _ in range(_K):
        m = jnp.max(work, axis=1, keepdims=True)
        sel = jnp.where(work == m, iota, _NT)
        idx = jnp.min(sel, axis=1, keepdims=True)
        cols.append(idx)
        work = jnp.where(iota == idx, -jnp.inf, work)
    topk_ref[...] = jnp.concatenate(cols, axis=1)
    qp_ref[...] = qe_ref[...] @ wpa_ref[...] + bp_ref[...]


def _tc_types(qt_emb, q_emb, w1, b1, w2, b2, ctt, wpa, bp):
    grid = (_B // _BM,)
    return pl.pallas_call(
        _tc_types_body,
        grid=grid,
        in_specs=[
            pl.BlockSpec((_BM, _D), lambda i: (i, 0)),
            pl.BlockSpec((_BM, _D), lambda i: (i, 0)),
            pl.BlockSpec((_D, _T), lambda i: (0, 0)),
            pl.BlockSpec((1, _T), lambda i: (0, 0)),
            pl.BlockSpec((_T, _T), lambda i: (0, 0)),
            pl.BlockSpec((1, _T), lambda i: (0, 0)),
            pl.BlockSpec((_T, _NT), lambda i: (0, 0)),
            pl.BlockSpec((_D, _D), lambda i: (0, 0)),
            pl.BlockSpec((1, _D), lambda i: (0, 0)),
        ],
        out_specs=[
            pl.BlockSpec((_BM, _NT), lambda i: (i, 0)),
            pl.BlockSpec((_BM, _K), lambda i: (i, 0)),
            pl.BlockSpec((_BM, _D), lambda i: (i, 0)),
        ],
        out_shape=[
            jax.ShapeDtypeStruct((_B, _NT), jnp.float32),
            jax.ShapeDtypeStruct((_B, _K), jnp.int32),
            jax.ShapeDtypeStruct((_B, _D), jnp.float32),
        ],
    )(qt_emb, q_emb, w1, b1, w2, b2, ctt, wpa, bp)


def _tc_ctype_proj_body(ct_ref, wpb_ref, o_ref):
    o_ref[...] = jnp.dot(ct_ref[...], wpb_ref[...],
                         preferred_element_type=jnp.float32)


def _tc_ctype_proj(ctype_table, wpb):
    return pl.pallas_call(
        _tc_ctype_proj_body,
        out_shape=jax.ShapeDtypeStruct((_NT, _D), jnp.float32),
    )(ctype_table, wpb)


def _tc_project_body(cg_ref, qp_ref, o_ref):
    o_ref[...] = jnp.tanh(qp_ref[...][:, None, :]
                          + cg_ref[...].reshape(_BM, _K, _D))


def _tc_project(cgath, q_proj):
    grid = (_B // _BM,)
    return pl.pallas_call(
        _tc_project_body,
        grid=grid,
        in_specs=[
            pl.BlockSpec((_BM * _K, _D), lambda i: (i, 0)),
            pl.BlockSpec((_BM, _D), lambda i: (i, 0)),
        ],
        out_specs=pl.BlockSpec((_BM, _K, _D), lambda i: (i, 0, 0)),
        out_shape=jax.ShapeDtypeStruct((_B, _K, _D), jnp.float32),
    )(cgath, q_proj)


def kernel(product_table, qtype_table, ctype_table, W1, b1, W2, b2, Wp, bp,
           query_ids, query_types):
    qids2d = query_ids.astype(jnp.int32).reshape(-1, 128)
    qtys2d = query_types.astype(jnp.int32).reshape(-1, 128)

    ttable_pad = jnp.zeros((_NT, _D), jnp.float32).at[:, :_T].set(qtype_table)
    w1_pad = jnp.zeros((_D, _T), jnp.float32).at[:_T].set(W1)

    q_emb, qt_emb = _sc_gather_embeddings(product_table, ttable_pad,
                                          qids2d, qtys2d)

    ctt = ctype_table.T
    wpa = Wp[:_D]
    wpb = Wp[_D:]
    sims, topk_idx, q_proj = _tc_types(
        qt_emb, q_emb, w1_pad, b1.reshape(1, _T), W2, b2.reshape(1, _T),
        ctt, wpa, bp.reshape(1, _D))

    ctype_proj = _tc_ctype_proj(ctype_table, wpb)
    cgath = _sc_gather_ctypes(ctype_proj, topk_idx.reshape(-1, 128))
    projected = _tc_project(cgath, q_proj)
    return projected, topk_idx, sims


# speedup vs baseline: 1.9029x; 1.0000x over previous
import functools

import jax
import jax.numpy as jnp
from jax import lax
from jax.experimental import pallas as pl
from jax.experimental.pallas import tpu as pltpu
from jax.experimental.pallas import tpu_sc as plsc

_B = 16384
_D = 128
_T = 64
_NT = 1000
_K = 8
_BM = 256


def _sc_gather_embeddings(ptable, ttable, qids2d, qtys2d):
    mesh = plsc.VectorSubcoreMesh(core_axis_name="c", subcore_axis_name="s")
    nw = mesh.num_cores * mesh.num_subcores
    rows = qids2d.shape[0]
    rpw = rows // nw

    @functools.partial(
        pl.kernel,
        out_type=(
            jax.ShapeDtypeStruct((_B, _D), jnp.float32),
            jax.ShapeDtypeStruct((_B, _D), jnp.float32),
        ),
        mesh=mesh,
        scratch_types=[
            pltpu.VMEM((rpw, 128), jnp.int32),
            pltpu.VMEM((rpw, 128), jnp.int32),
            pltpu.VMEM((2, 128, _D), jnp.float32),
            pltpu.VMEM((2, 128, _D), jnp.float32),
            pltpu.SemaphoreType.DMA,
        ],
    )
    def k(pt, tt, qi, qy, oq, ot, qi_v, qy_v, qbuf, tbuf, sem):
        wid = lax.axis_index("s") * mesh.num_cores + lax.axis_index("c")
        rbase = wid * rpw
        pltpu.sync_copy(qi.at[pl.ds(rbase, rpw)], qi_v)
        pltpu.sync_copy(qy.at[pl.ds(rbase, rpw)], qy_v)
        for j in range(rpw):
            s = j % 2
            cq = pltpu.async_copy(pt.at[qi_v.at[j]], qbuf.at[s], sem)
            ct = pltpu.async_copy(tt.at[qy_v.at[j]], tbuf.at[s], sem)
            cq.wait()
            ct.wait()
            obase = (rbase + j) * 128
            pltpu.sync_copy(qbuf.at[s], oq.at[pl.ds(obase, 128)])
            pltpu.sync_copy(tbuf.at[s], ot.at[pl.ds(obase, 128)])

    return k(ptable, ttable, qids2d, qtys2d)


def _sc_gather_ctypes(ttable, idx2d):
    mesh = plsc.VectorSubcoreMesh(core_axis_name="c", subcore_axis_name="s")
    nw = mesh.num_cores * mesh.num_subcores
    rows = idx2d.shape[0]
    rpw = rows // nw
    nbk = rows * 128

    @functools.partial(
        pl.kernel,
        out_type=jax.ShapeDtypeStruct((nbk, _D), jnp.float32),
        mesh=mesh,
        scratch_types=[
            pltpu.VMEM((rpw, 128), jnp.int32),
            pltpu.VMEM((4, 128, _D), jnp.float32),
            pltpu.SemaphoreType.DMA,
            pltpu.SemaphoreType.DMA,
        ],
    )
    def k(tt, idx, out, idx_v, buf, gsem, wsem):
        wid = lax.axis_index("s") * mesh.num_cores + lax.axis_index("c")
        rbase = wid * rpw
        pltpu.sync_copy(idx.at[pl.ds(rbase, rpw)], idx_v)

        def group(g, carry):
            cps = [pltpu.async_copy(tt.at[idx_v.at[g * 4 + b]], buf.at[b],
                                    gsem) for b in range(4)]
            wps = []
            for b in range(4):
                cps[b].wait()
                wps.append(pltpu.async_copy(
                    buf.at[b], out.at[pl.ds((rbase + g * 4 + b) * 128, 128)],
                    wsem))
            for w in wps:
                w.wait()
            return carry

        lax.fori_loop(0, rpw // 4, group, 0)

    return k(ttable, idx2d)


def _tc_types_body(qt_ref, qe_ref, w1_ref, b1_ref, w2_ref, b2_ref, ctt_ref,
                   wpa_ref, bp_ref, sims_ref, topk_ref, qp_ref):
    qt = qt_ref[...]
    h = jnp.maximum(qt @ w1_ref[...] + b1_ref[...], 0.0)
    cb = h @ w2_ref[...] + b2_ref[...]
    s = jnp.dot(cb, ctt_ref[...], preferred_element_type=jnp.float32)
    sims_ref[...] = s
    iota = lax.broadcasted_iota(jnp.int32, s.shape, 1)
    work = s
    cols = []
    for _ in range(_K):
        m = jnp.max(work, axis=1, keepdims=True)
        sel = jnp.where(work == m, iota, _NT)
        idx = jnp.min(sel, axis=1, keepdims=True)
        cols.append(idx)
        work = jnp.where(iota == idx, -jnp.inf, work)
    topk_ref[...] = jnp.concatenate(cols, axis=1)
    qp_ref[...] = qe_ref[...] @ wpa_ref[...] + bp_ref[...]


def _tc_types(qt_emb, q_emb, w1, b1, w2, b2, ctt, wpa, bp):
    grid = (_B // _BM,)
    return pl.pallas_call(
        _tc_types_body,
        grid=grid,
        in_specs=[
            pl.BlockSpec((_BM, _D), lambda i: (i, 0)),
            pl.BlockSpec((_BM, _D), lambda i: (i, 0)),
            pl.BlockSpec((_D, _T), lambda i: (0, 0)),
            pl.BlockSpec((1, _T), lambda i: (0, 0)),
            pl.BlockSpec((_T, _T), lambda i: (0, 0)),
            pl.BlockSpec((1, _T), lambda i: (0, 0)),
            pl.BlockSpec((_T, _NT), lambda i: (0, 0)),
            pl.BlockSpec((_D, _D), lambda i: (0, 0)),
            pl.BlockSpec((1, _D), lambda i: (0, 0)),
        ],
        out_specs=[
            pl.BlockSpec((_BM, _NT), lambda i: (i, 0)),
            pl.BlockSpec((_BM, _K), lambda i: (i, 0)),
            pl.BlockSpec((_BM, _D), lambda i: (i, 0)),
        ],
        out_shape=[
            jax.ShapeDtypeStruct((_B, _NT), jnp.float32),
            jax.ShapeDtypeStruct((_B, _K), jnp.int32),
            jax.ShapeDtypeStruct((_B, _D), jnp.float32),
        ],
    )(qt_emb, q_emb, w1, b1, w2, b2, ctt, wpa, bp)


def _tc_ctype_proj_body(ct_ref, wpb_ref, o_ref):
    o_ref[...] = jnp.dot(ct_ref[...], wpb_ref[...],
                         preferred_element_type=jnp.float32)


def _tc_ctype_proj(ctype_table, wpb):
    return pl.pallas_call(
        _tc_ctype_proj_body,
        out_shape=jax.ShapeDtypeStruct((_NT, _D), jnp.float32),
    )(ctype_table, wpb)


def _tc_project_body(cg_ref, qp_ref, o_ref):
    o_ref[...] = jnp.tanh(qp_ref[...][:, None, :]
                          + cg_ref[...].reshape(_BM, _K, _D))


def _tc_project(cgath, q_proj):
    grid = (_B // _BM,)
    return pl.pallas_call(
        _tc_project_body,
        grid=grid,
        in_specs=[
            pl.BlockSpec((_BM * _K, _D), lambda i: (i, 0)),
            pl.BlockSpec((_BM, _D), lambda i: (i, 0)),
        ],
        out_specs=pl.BlockSpec((_BM, _K, _D), lambda i: (i, 0, 0)),
        out_shape=jax.ShapeDtypeStruct((_B, _K, _D), jnp.float32),
    )(cgath, q_proj)


def kernel(product_table, qtype_table, ctype_table, W1, b1, W2, b2, Wp, bp,
           query_ids, query_types):
    qids2d = query_ids.astype(jnp.int32).reshape(-1, 128)
    qtys2d = query_types.astype(jnp.int32).reshape(-1, 128)

    ttable_pad = jnp.zeros((_NT, _D), jnp.float32).at[:, :_T].set(qtype_table)
    w1_pad = jnp.zeros((_D, _T), jnp.float32).at[:_T].set(W1)

    q_emb, qt_emb = _sc_gather_embeddings(product_table, ttable_pad,
                                          qids2d, qtys2d)

    ctt = ctype_table.T
    wpa = Wp[:_D]
    wpb = Wp[_D:]
    sims, topk_idx, q_proj = _tc_types(
        qt_emb, q_emb, w1_pad, b1.reshape(1, _T), W2, b2.reshape(1, _T),
        ctt, wpa, bp.reshape(1, _D))

    ctype_proj = _tc_ctype_proj(ctype_table, wpb)
    cgath = _sc_gather_ctypes(ctype_proj, topk_idx.reshape(-1, 128))
    projected = _tc_project(cgath, q_proj)
    return projected, topk_idx, sims


# speedup vs baseline: 4.1985x; 2.2063x over previous
import functools

import jax
import jax.numpy as jnp
from jax import lax
from jax.experimental import pallas as pl
from jax.experimental.pallas import tpu as pltpu
from jax.experimental.pallas import tpu_sc as plsc

_B = 16384
_D = 128
_T = 64
_NT = 1000
_K = 8
_BM = 256


def _sc_gather_embeddings(ptable, ttable, qids2d, qtys2d):
    mesh = plsc.VectorSubcoreMesh(core_axis_name="c", subcore_axis_name="s")
    nw = mesh.num_cores * mesh.num_subcores
    rows = qids2d.shape[0]
    rpw = rows // nw

    @functools.partial(
        pl.kernel,
        out_type=(
            jax.ShapeDtypeStruct((_B, _D), jnp.float32),
            jax.ShapeDtypeStruct((_B, _D), jnp.float32),
        ),
        mesh=mesh,
        scratch_types=[
            pltpu.VMEM((rpw, 128), jnp.int32),
            pltpu.VMEM((rpw, 128), jnp.int32),
            pltpu.VMEM((2, 128, _D), jnp.float32),
            pltpu.VMEM((2, 128, _D), jnp.float32),
            pltpu.SemaphoreType.DMA,
        ],
    )
    def k(pt, tt, qi, qy, oq, ot, qi_v, qy_v, qbuf, tbuf, sem):
        wid = lax.axis_index("s") * mesh.num_cores + lax.axis_index("c")
        rbase = wid * rpw
        pltpu.sync_copy(qi.at[pl.ds(rbase, rpw)], qi_v)
        pltpu.sync_copy(qy.at[pl.ds(rbase, rpw)], qy_v)
        for j in range(rpw):
            s = j % 2
            cq = pltpu.async_copy(pt.at[qi_v.at[j]], qbuf.at[s], sem)
            ct = pltpu.async_copy(tt.at[qy_v.at[j]], tbuf.at[s], sem)
            cq.wait()
            ct.wait()
            obase = (rbase + j) * 128
            pltpu.sync_copy(qbuf.at[s], oq.at[pl.ds(obase, 128)])
            pltpu.sync_copy(tbuf.at[s], ot.at[pl.ds(obase, 128)])

    return k(ptable, ttable, qids2d, qtys2d)


def _sc_gather_ctypes(ttable, idx2d):
    mesh = plsc.VectorSubcoreMesh(core_axis_name="c", subcore_axis_name="s")
    nw = mesh.num_cores * mesh.num_subcores
    rows = idx2d.shape[0]
    rpw = rows // nw
    nbk = rows * 128

    @functools.partial(
        pl.kernel,
        out_type=jax.ShapeDtypeStruct((nbk, _D), jnp.float32),
        mesh=mesh,
        scratch_types=[
            pltpu.VMEM((rpw, 128), jnp.int32),
            pltpu.VMEM((4, 128, _D), jnp.float32),
            pltpu.VMEM_SHARED((_NT, _D), jnp.float32),
            pltpu.SemaphoreType.DMA,
            pltpu.SemaphoreType.DMA,
        ],
    )
    def k(tt, idx, out, idx_v, buf, shared, gsem, wsem):
        s = lax.axis_index("s")
        wid = s * mesh.num_cores + lax.axis_index("c")
        rbase = wid * rpw

        @pl.when(s == 0)
        def _stage_table():
            pltpu.sync_copy(tt, shared)

        pltpu.sync_copy(idx.at[pl.ds(rbase, rpw)], idx_v)
        plsc.subcore_barrier()

        def group(g, carry):
            cps = [pltpu.async_copy(shared.at[idx_v.at[g * 4 + b]], buf.at[b],
                                    gsem) for b in range(4)]
            wps = []
            for b in range(4):
                cps[b].wait()
                wps.append(pltpu.async_copy(
                    buf.at[b], out.at[pl.ds((rbase + g * 4 + b) * 128, 128)],
                    wsem))
            for w in wps:
                w.wait()
            return carry

        lax.fori_loop(0, rpw // 4, group, 0)

    return k(ttable, idx2d)


def _tc_types_body(qt_ref, qe_ref, w1_ref, b1_ref, w2_ref, b2_ref, ctt_ref,
                   wpa_ref, bp_ref, sims_ref, topk_ref, qp_ref):
    qt = qt_ref[...]
    h = jnp.maximum(qt @ w1_ref[...] + b1_ref[...], 0.0)
    cb = h @ w2_ref[...] + b2_ref[...]
    s = jnp.dot(cb, ctt_ref[...], preferred_element_type=jnp.float32)
    sims_ref[...] = s
    iota = lax.broadcasted_iota(jnp.int32, s.shape, 1)
    work = s
    cols = []
    for _ in range(_K):
        m = jnp.max(work, axis=1, keepdims=True)
        sel = jnp.where(work == m, iota, _NT)
        idx = jnp.min(sel, axis=1, keepdims=True)
        cols.append(idx)
        work = jnp.where(iota == idx, -jnp.inf, work)
    topk_ref[...] = jnp.concatenate(cols, axis=1)
    qp_ref[...] = qe_ref[...] @ wpa_ref[...] + bp_ref[...]


def _tc_types(qt_emb, q_emb, w1, b1, w2, b2, ctt, wpa, bp):
    grid = (_B // _BM,)
    return pl.pallas_call(
        _tc_types_body,
        grid=grid,
        in_specs=[
            pl.BlockSpec((_BM, _D), lambda i: (i, 0)),
            pl.BlockSpec((_BM, _D), lambda i: (i, 0)),
            pl.BlockSpec((_D, _T), lambda i: (0, 0)),
            pl.BlockSpec((1, _T), lambda i: (0, 0)),
            pl.BlockSpec((_T, _T), lambda i: (0, 0)),
            pl.BlockSpec((1, _T), lambda i: (0, 0)),
            pl.BlockSpec((_T, _NT), lambda i: (0, 0)),
            pl.BlockSpec((_D, _D), lambda i: (0, 0)),
            pl.BlockSpec((1, _D), lambda i: (0, 0)),
        ],
        out_specs=[
            pl.BlockSpec((_BM, _NT), lambda i: (i, 0)),
            pl.BlockSpec((_BM, _K), lambda i: (i, 0)),
            pl.BlockSpec((_BM, _D), lambda i: (i, 0)),
        ],
        out_shape=[
            jax.ShapeDtypeStruct((_B, _NT), jnp.float32),
            jax.ShapeDtypeStruct((_B, _K), jnp.int32),
            jax.ShapeDtypeStruct((_B, _D), jnp.float32),
        ],
    )(qt_emb, q_emb, w1, b1, w2, b2, ctt, wpa, bp)


def _tc_ctype_proj_body(ct_ref, wpb_ref, o_ref):
    o_ref[...] = jnp.dot(ct_ref[...], wpb_ref[...],
                         preferred_element_type=jnp.float32)


def _tc_ctype_proj(ctype_table, wpb):
    return pl.pallas_call(
        _tc_ctype_proj_body,
        out_shape=jax.ShapeDtypeStruct((_NT, _D), jnp.float32),
    )(ctype_table, wpb)


def _tc_project_body(cg_ref, qp_ref, o_ref):
    o_ref[...] = jnp.tanh(qp_ref[...][:, None, :]
                          + cg_ref[...].reshape(_BM, _K, _D))


def _tc_project(cgath, q_proj):
    grid = (_B // _BM,)
    return pl.pallas_call(
        _tc_project_body,
        grid=grid,
        in_specs=[
            pl.BlockSpec((_BM * _K, _D), lambda i: (i, 0)),
            pl.BlockSpec((_BM, _D), lambda i: (i, 0)),
        ],
        out_specs=pl.BlockSpec((_BM, _K, _D), lambda i: (i, 0, 0)),
        out_shape=jax.ShapeDtypeStruct((_B, _K, _D), jnp.float32),
    )(cgath, q_proj)


def kernel(product_table, qtype_table, ctype_table, W1, b1, W2, b2, Wp, bp,
           query_ids, query_types):
    qids2d = query_ids.astype(jnp.int32).reshape(-1, 128)
    qtys2d = query_types.astype(jnp.int32).reshape(-1, 128)

    ttable_pad = jnp.zeros((_NT, _D), jnp.float32).at[:, :_T].set(qtype_table)
    w1_pad = jnp.zeros((_D, _T), jnp.float32).at[:_T].set(W1)

    q_emb, qt_emb = _sc_gather_embeddings(product_table, ttable_pad,
                                          qids2d, qtys2d)

    ctt = ctype_table.T
    wpa = Wp[:_D]
    wpb = Wp[_D:]
    sims, topk_idx, q_proj = _tc_types(
        qt_emb, q_emb, w1_pad, b1.reshape(1, _T), W2, b2.reshape(1, _T),
        ctt, wpa, bp.reshape(1, _D))

    ctype_proj = _tc_ctype_proj(ctype_table, wpb)
    cgath = _sc_gather_ctypes(ctype_proj, topk_idx.reshape(-1, 128))
    projected = _tc_project(cgath, q_proj)
    return projected, topk_idx, sims
